# Initial kernel scaffold; baseline (speedup 1.0000x reference)
import jax
import jax.numpy as jnp
from jax.experimental import pallas as pl


def kernel(embeddings, edge_index):
    raise NotImplementedError("write your pallas kernel here")


# speedup vs baseline: 17.6975x; 17.6975x over previous
import functools

import jax
import jax.numpy as jnp
from jax import lax
from jax.experimental import pallas as pl
from jax.experimental.pallas import tpu as pltpu
from jax.experimental.pallas import tpu_sc as plsc

N_NODES = 10000
N_EDGES = 320000
D_FEAT = 128

NC = 2
NS = 16
NW = NC * NS

CH = 128
NCHUNKS = N_EDGES // CH
CHUNKS_Q, CHUNKS_R = divmod(NCHUNKS, NW)

NPAD = 10240
ROWS_PER_TILE = NPAD // NS
DRAIN_BLK = 128
N_DRAIN = ROWS_PER_TILE // DRAIN_BLK

_mesh = plsc.VectorSubcoreMesh(core_axis_name="c", subcore_axis_name="s",
                               num_cores=NC, num_subcores=NS)


def _worker_chunks(wid):
    start = wid * CHUNKS_Q + jnp.minimum(wid, CHUNKS_R)
    count = CHUNKS_Q + (wid < CHUNKS_R).astype(jnp.int32)
    return start, count


@functools.partial(
    pl.kernel,
    out_type=(
        jax.ShapeDtypeStruct((NC, NPAD), jnp.float32),
        jax.ShapeDtypeStruct((NC, NPAD), jnp.float32),
    ),
    mesh=_mesh,
    scratch_types=[
        pltpu.VMEM((1, CH), jnp.int32),
        pltpu.VMEM((1, CH), jnp.int32),
        pltpu.VMEM((CH,), jnp.float32),
        pltpu.VMEM((ROWS_PER_TILE,), jnp.float32),
        pltpu.VMEM_SHARED((NPAD,), jnp.float32),
        pltpu.VMEM_SHARED((NPAD,), jnp.float32),
    ],
)
def _deg_kernel(edge_hbm, degr_hbm, degc_hbm,
                ridx, cidx, ones_v, bounce, degr_sh, degc_sh):
    cid = lax.axis_index("c")
    sid = lax.axis_index("s")
    wid = cid * NS + sid

    def fill16(i, _):
        bounce[pl.ds(i * 16, 16)] = jnp.zeros((16,), jnp.float32)
        return 0
    lax.fori_loop(0, ROWS_PER_TILE // 16, fill16, 0)
    for i in range(CH // 16):
        ones_v[pl.ds(i * 16, 16)] = jnp.ones((16,), jnp.float32)

    tile_base = sid * ROWS_PER_TILE
    pltpu.sync_copy(bounce, degr_sh.at[pl.ds(tile_base, ROWS_PER_TILE)])
    pltpu.sync_copy(bounce, degc_sh.at[pl.ds(tile_base, ROWS_PER_TILE)])
    plsc.subcore_barrier()

    start, count = _worker_chunks(wid)

    def body(j, _):
        base = (start + j) * CH
        pltpu.sync_copy(edge_hbm.at[0, pl.ds(base, CH)], ridx.at[0])
        pltpu.sync_copy(edge_hbm.at[1, pl.ds(base, CH)], cidx.at[0])
        pltpu.sync_copy(ones_v, degr_sh.at[ridx.at[0]], add=True)
        pltpu.sync_copy(ones_v, degc_sh.at[cidx.at[0]], add=True)
        return 0
    lax.fori_loop(0, count, body, 0)

    plsc.subcore_barrier()
    pltpu.sync_copy(degr_sh.at[pl.ds(tile_base, ROWS_PER_TILE)], bounce)
    pltpu.sync_copy(bounce, degr_hbm.at[cid, pl.ds(tile_base, ROWS_PER_TILE)])
    pltpu.sync_copy(degc_sh.at[pl.ds(tile_base, ROWS_PER_TILE)], bounce)
    pltpu.sync_copy(bounce, degc_hbm.at[cid, pl.ds(tile_base, ROWS_PER_TILE)])


@functools.partial(
    pl.kernel,
    out_type=(
        jax.ShapeDtypeStruct((NC, NPAD, D_FEAT), jnp.float32),
    ),
    mesh=_mesh,
    scratch_types=[
        pltpu.VMEM((1, CH), jnp.int32),
        pltpu.VMEM((1, CH), jnp.int32),
        pltpu.VMEM((CH, D_FEAT), jnp.float32),
        pltpu.VMEM_SHARED((NPAD, D_FEAT), jnp.float32),
        pltpu.SemaphoreType.DMA,
    ],
)
def _spmm_kernel(scaled_hbm, edge_hbm, out_hbm,
                 ridx, cidx, gbuf, acc_sh, gsem):
    cid = lax.axis_index("c")
    sid = lax.axis_index("s")
    wid = cid * NS + sid

    def fill16(i, _):
        r = i // (D_FEAT // 16)
        k = i % (D_FEAT // 16)
        gbuf[r, pl.ds(k * 16, 16)] = jnp.zeros((16,), jnp.float32)
        return 0
    lax.fori_loop(0, DRAIN_BLK * (D_FEAT // 16), fill16, 0)

    tile_base = sid * ROWS_PER_TILE
    for k in range(N_DRAIN):
        pltpu.sync_copy(gbuf, acc_sh.at[pl.ds(tile_base + k * DRAIN_BLK, DRAIN_BLK)])
    plsc.subcore_barrier()

    start, count = _worker_chunks(wid)

    def body(j, _):
        base = (start + j) * CH
        pltpu.sync_copy(edge_hbm.at[1, pl.ds(base, CH)], cidx.at[0])
        pltpu.async_copy(scaled_hbm.at[cidx.at[0]], gbuf, gsem).wait()
        pltpu.sync_copy(edge_hbm.at[0, pl.ds(base, CH)], ridx.at[0])
        pltpu.sync_copy(gbuf, acc_sh.at[ridx.at[0]], add=True)
        return 0
    lax.fori_loop(0, count, body, 0)

    plsc.subcore_barrier()
    for k in range(N_DRAIN):
        rows = pl.ds(tile_base + k * DRAIN_BLK, DRAIN_BLK)
        pltpu.sync_copy(acc_sh.at[rows], gbuf)
        pltpu.sync_copy(gbuf, out_hbm.at[cid, rows])


def _prep_body(degr_ref, degc_ref, emb_ref, scaled_ref, rrow_ref):
    degr = degr_ref[0] + degr_ref[1]
    degc = degc_ref[0] + degc_ref[1]
    rrow_ref[...] = lax.rsqrt(jnp.maximum(degr, 1.0))
    rcol = lax.rsqrt(jnp.maximum(degc, 1.0))
    rcol_n = rcol.reshape(NPAD)[:N_NODES]
    scaled_ref[...] = emb_ref[...] * rcol_n[:, None]


def _finish_body(parts_ref, rrow_ref, out_ref):
    acc = parts_ref[0, :N_NODES, :] + parts_ref[1, :N_NODES, :]
    rrow = rrow_ref[...].reshape(NPAD)[:N_NODES]
    out_ref[...] = acc * rrow[:, None]


def kernel(embeddings, edge_index):
    degr_p, degc_p = _deg_kernel(edge_index)
    scaled, rrow = pl.pallas_call(
        _prep_body,
        out_shape=(
            jax.ShapeDtypeStruct((N_NODES, D_FEAT), jnp.float32),
            jax.ShapeDtypeStruct((NPAD // 128, 128), jnp.float32),
        ),
    )(degr_p.reshape(NC, NPAD // 128, 128),
      degc_p.reshape(NC, NPAD // 128, 128),
      embeddings)
    (parts,) = _spmm_kernel(scaled, edge_index)
    out = pl.pallas_call(
        _finish_body,
        out_shape=jax.ShapeDtypeStruct((N_NODES, D_FEAT), jnp.float32),
    )(parts, rrow)
    return out
